# baseline (speedup 1.0000x reference)
import jax
import jax.numpy as jnp
from jax import lax
from jax.experimental import pallas as pl
from jax.experimental.pallas import tpu as pltpu

_N = 100000
_D = 128
_B = 2000


def _body(keys_ref, q_ref, out_ref, val_ref, pid_ref):
    pid = pl.program_id(0)

    @pl.when(pid == 0)
    def _init():
        val_ref[...] = jnp.full((_B,), jnp.inf, jnp.float32)
        pid_ref[...] = jnp.zeros((_B,), jnp.int32)

    x = jnp.abs(keys_ref[...] - q_ref[...])
    s = jnp.sum(x, axis=1)

    upd = s < val_ref[...]
    val_ref[...] = jnp.where(upd, s, val_ref[...])
    pid_ref[...] = jnp.where(upd, jnp.full((_B,), pid, jnp.int32),
                             pid_ref[...])

    @pl.when(pid == pl.num_programs(0) - 1)
    def _emit():
        val = val_ref[...]
        m = jnp.min(val)
        rows = pid_ref[...] * _B + lax.broadcasted_iota(jnp.int32, (_B,), 0)
        out_ref[0] = jnp.min(jnp.where(val == m, rows, jnp.int32(_N)))


def kernel(keys, query):
    out = pl.pallas_call(
        _body,
        grid=(_N // _B,),
        in_specs=[
            pl.BlockSpec((_B, _D), lambda i: (i, 0)),
            pl.BlockSpec((1, _D), lambda i: (0, 0)),
        ],
        out_specs=pl.BlockSpec(memory_space=pltpu.SMEM),
        out_shape=jax.ShapeDtypeStruct((1,), jnp.int32),
        scratch_shapes=[
            pltpu.VMEM((_B,), jnp.float32),
            pltpu.VMEM((_B,), jnp.int32),
        ],
    )(keys, query)
    return out[0]


# speedup vs baseline: 2.7402x; 2.7402x over previous
import jax
import jax.numpy as jnp
from jax import lax
from jax.experimental import pallas as pl
from jax.experimental.pallas import tpu as pltpu

_N = 100000
_D = 128
_B = 2000


def _body(keys_ref, q_ref, out_ref, val_ref, pid_ref):
    pid = pl.program_id(0)

    @pl.when(pid == 0)
    def _init():
        val_ref[...] = jnp.full((_B, 1), jnp.inf, jnp.float32)
        pid_ref[...] = jnp.zeros((_B, 1), jnp.int32)

    x = jnp.abs(keys_ref[...] - q_ref[...])
    s = jnp.sum(x, axis=1, keepdims=True)
    upd = s < val_ref[...]
    val_ref[...] = jnp.where(upd, s, val_ref[...])
    pid_ref[...] = jnp.where(upd, jnp.full((_B, 1), pid, jnp.int32),
                             pid_ref[...])

    @pl.when(pid == pl.num_programs(0) - 1)
    def _emit():
        val = val_ref[...]
        m = jnp.min(val)
        rows = (pid_ref[...] * _B
                + lax.broadcasted_iota(jnp.int32, (_B, 1), 0))
        out_ref[0] = jnp.min(jnp.where(val == m, rows, jnp.int32(_N)))


def kernel(keys, query):
    out = pl.pallas_call(
        _body,
        grid=(_N // _B,),
        in_specs=[
            pl.BlockSpec((_B, _D), lambda i: (i, 0)),
            pl.BlockSpec((1, _D), lambda i: (0, 0)),
        ],
        out_specs=pl.BlockSpec(memory_space=pltpu.SMEM),
        out_shape=jax.ShapeDtypeStruct((1,), jnp.int32),
        scratch_shapes=[
            pltpu.VMEM((_B, 1), jnp.float32),
            pltpu.VMEM((_B, 1), jnp.int32),
        ],
    )(keys, query)
    return out[0]


# speedup vs baseline: 4.6952x; 1.7134x over previous
import jax
import jax.numpy as jnp
from jax import lax
from jax.experimental import pallas as pl
from jax.experimental.pallas import tpu as pltpu

_N = 100000
_D = 128
_B = 10000


def _body(keys_ref, q_ref, out_ref, val_ref, pid_ref):
    pid = pl.program_id(0)

    @pl.when(pid == 0)
    def _init():
        val_ref[...] = jnp.full((_B, 1), jnp.inf, jnp.float32)
        pid_ref[...] = jnp.zeros((_B, 1), jnp.int32)

    x = jnp.abs(keys_ref[...] - q_ref[...])
    s = jnp.sum(x, axis=1, keepdims=True)
    upd = s < val_ref[...]
    val_ref[...] = jnp.where(upd, s, val_ref[...])
    pid_ref[...] = jnp.where(upd, jnp.full((_B, 1), pid, jnp.int32),
                             pid_ref[...])

    @pl.when(pid == pl.num_programs(0) - 1)
    def _emit():
        val = val_ref[...]
        m = jnp.min(val)
        rows = (pid_ref[...] * _B
                + lax.broadcasted_iota(jnp.int32, (_B, 1), 0))
        out_ref[0] = jnp.min(jnp.where(val == m, rows, jnp.int32(_N)))


def kernel(keys, query):
    out = pl.pallas_call(
        _body,
        grid=(_N // _B,),
        in_specs=[
            pl.BlockSpec((_B, _D), lambda i: (i, 0)),
            pl.BlockSpec((1, _D), lambda i: (0, 0)),
        ],
        out_specs=pl.BlockSpec(memory_space=pltpu.SMEM),
        out_shape=jax.ShapeDtypeStruct((1,), jnp.int32),
        scratch_shapes=[
            pltpu.VMEM((_B, 1), jnp.float32),
            pltpu.VMEM((_B, 1), jnp.int32),
        ],
    )(keys, query)
    return out[0]
